# baseline (speedup 1.0000x reference)
import functools
import jax
import jax.numpy as jnp
from jax import lax
from jax.experimental import pallas as pl
from jax.experimental.pallas import tpu as pltpu
from jax.experimental.pallas import tpu_sc as plsc

_EPS = 1e-08
_B, _S, _D = 4, 2048, 768
_NUM_CLASSES_SEEN = 10
_TOP_K = 3
_N_DESC = 5

_TOTAL_ROWS = _B * _S
_TC_ROWS = 3 * _S
_SC_ROWS = _TOTAL_ROWS - _TC_ROWS
_NW = 16
_ROWS_PER_W = _SC_ROWS // _NW
_SC_CHUNK = 32
_SC_NCHUNK = _ROWS_PER_W // _SC_CHUNK
_VECS = _D // 16

_NSTEPS = 3
_ROWS_TC = _TC_ROWS // _NSTEPS


def _sc_body(x_hbm, out_hbm, buf0, buf1, acc, sem0, sem1):
    wid = lax.axis_index("s")
    base = _TC_ROWS + wid * _ROWS_PER_W

    bufs = (buf0, buf1)
    sems = (sem0, sem1)

    cps = {}
    for k in range(2):
        cps[k] = pltpu.async_copy(
            x_hbm.at[pl.ds(base + k * _SC_CHUNK, _SC_CHUNK)],
            bufs[k], sems[k])

    for j in range(_VECS):
        acc[0, pl.ds(16 * j, 16)] = jnp.zeros((16,), jnp.float32)

    for k in range(_SC_NCHUNK):
        b = k % 2
        cps[k].wait()

        def _row(r, _, buf=bufs[b]):
            for j in range(_VECS):
                plsc.addupdate(acc.at[0, pl.ds(16 * j, 16)],
                               buf[r, pl.ds(16 * j, 16)])
            return 0

        lax.fori_loop(0, _SC_CHUNK, _row, 0, unroll=4)
        if k + 2 < _SC_NCHUNK:
            cps[k + 2] = pltpu.async_copy(
                x_hbm.at[pl.ds(base + (k + 2) * _SC_CHUNK, _SC_CHUNK)],
                bufs[b], sems[b])

    pltpu.sync_copy(acc, out_hbm.at[pl.ds(wid, 1)])


@functools.partial(
    pl.kernel,
    out_type=jax.ShapeDtypeStruct((_NW, _D), jnp.float32),
    mesh=plsc.VectorSubcoreMesh(core_axis_name="c", subcore_axis_name="s",
                                num_cores=1),
    scratch_types=[
        pltpu.VMEM((_SC_CHUNK, _D), jnp.float32),
        pltpu.VMEM((_SC_CHUNK, _D), jnp.float32),
        pltpu.VMEM((1, _D), jnp.float32),
        pltpu.SemaphoreType.DMA,
        pltpu.SemaphoreType.DMA,
    ],
)
def _sc_partial_sums(x_hbm, out_hbm, buf0, buf1, acc, sem0, sem1):
    _sc_body(x_hbm, out_hbm, buf0, buf1, acc, sem0, sem1)


def _tc_body(x_ref, sc_ref, ck_ref, desc_ref, w_ref, g_ref, b_ref, t_ref,
             out_ref, acc_ref, dp_ref):
    i = pl.program_id(0)

    if _ROWS_TC <= _S:
        blks_per_batch = _S // _ROWS_TC
        partial = jnp.sum(x_ref[...], axis=0, keepdims=True)
        b = i // blks_per_batch

        @pl.when(i % blks_per_batch == 0)
        def _init():
            acc_ref[pl.ds(b, 1), :] = partial

        @pl.when(i % blks_per_batch != 0)
        def _accum():
            acc_ref[pl.ds(b, 1), :] = acc_ref[pl.ds(b, 1), :] + partial
    else:
        bpb = _ROWS_TC // _S
        partial = jnp.sum(x_ref[...].reshape(bpb, _S, _D), axis=1)
        for k in range(_NSTEPS):
            @pl.when(i == k)
            def _store(k=k):
                acc_ref[k * bpb:(k + 1) * bpb, :] = partial

    @pl.when(i == 0)
    def _dprime():
        dp_ref[0:_N_DESC, :] = jax.lax.dot_general(
            desc_ref[...], w_ref[...], (((1,), (1,)), ((), ())),
            preferred_element_type=jnp.float32)

    @pl.when(i == _NSTEPS - 1)
    def _epilogue():
        wpb = _NW // (_B - (_TC_ROWS // _S))
        sums = [acc_ref[0:(_TC_ROWS // _S), :]]
        for sb in range(_B - (_TC_ROWS // _S)):
            sums.append(jnp.sum(sc_ref[sb * wpb:(sb + 1) * wpb, :],
                                axis=0, keepdims=True))
        total = jnp.concatenate(sums, axis=0)

        mean = total * (1.0 / _S)
        xnorm = jnp.sqrt(jnp.sum(mean * mean, axis=1, keepdims=True))
        xn = mean / jnp.maximum(xnorm, _EPS)

        ck = ck_ref[...]
        cknorm = jnp.sqrt(jnp.sum(ck * ck, axis=1, keepdims=True))
        ckn = ck / jnp.maximum(cknorm, _EPS)

        sims = jax.lax.dot_general(
            xn, ckn, (((1,), (1,)), ((), ())),
            preferred_element_type=jnp.float32)

        t = t_ref[0, 0]

        col = jax.lax.broadcasted_iota(jnp.int32, (_B, _NUM_CLASSES_SEEN), 1)
        s = sims
        vals = []
        idxs = []
        for _ in range(_TOP_K):
            m = jnp.max(s, axis=1, keepdims=True)
            idx = jnp.min(jnp.where(s >= m, col, _NUM_CLASSES_SEEN + 1),
                          axis=1, keepdims=True)
            vals.append(m)
            idxs.append(idx)
            s = jnp.where(col == idx, -jnp.inf, s)

        exps = [jnp.exp((v - vals[0]) * t) for v in vals]
        denom = exps[0] + exps[1] + exps[2]
        ws = [e / denom for e in exps]

        dcol = jax.lax.broadcasted_iota(jnp.int32, (_B, _N_DESC), 1)
        dw = jnp.zeros((_B, _N_DESC), jnp.float32)
        for k in range(_TOP_K):
            didx = jax.lax.rem(idxs[k], _N_DESC)
            dw = dw + jnp.where(dcol == didx, ws[k], 0.0)

        proj = jax.lax.dot_general(
            dw, dp_ref[0:_N_DESC, :], (((1,), (0,)), ((), ())),
            preferred_element_type=jnp.float32)

        mu = jnp.mean(proj, axis=1, keepdims=True)
        ctr = proj - mu
        var = jnp.mean(ctr * ctr, axis=1, keepdims=True)
        ln = ctr * jax.lax.rsqrt(var + 1e-05) * g_ref[...] + b_ref[...]

        out_ref[:, 0, :] = ln


@jax.jit
def kernel(x_embed, prompt_key, task_key, desc_emb, W_proj, ln_gamma,
           ln_beta, temperature):
    del task_key
    xf = x_embed.reshape(_TOTAL_ROWS, _D)
    ck = prompt_key[:_NUM_CLASSES_SEEN]
    gamma = ln_gamma.reshape(1, _D)
    beta = ln_beta.reshape(1, _D)
    temp = temperature.reshape(1, 1)

    sc_part = _sc_partial_sums(xf)

    out = pl.pallas_call(
        _tc_body,
        grid=(_NSTEPS,),
        in_specs=[
            pl.BlockSpec((_ROWS_TC, _D), lambda i: (i, 0)),
            pl.BlockSpec((_NW, _D), lambda i: (0, 0)),
            pl.BlockSpec((_NUM_CLASSES_SEEN, _D), lambda i: (0, 0)),
            pl.BlockSpec((_N_DESC, _D), lambda i: (0, 0)),
            pl.BlockSpec((_D, _D), lambda i: (0, 0)),
            pl.BlockSpec((1, _D), lambda i: (0, 0)),
            pl.BlockSpec((1, _D), lambda i: (0, 0)),
            pl.BlockSpec((1, 1), lambda i: (0, 0)),
        ],
        out_specs=pl.BlockSpec((_B, 1, _D), lambda i: (0, 0, 0)),
        out_shape=jax.ShapeDtypeStruct((_B, 1, _D), jnp.float32),
        scratch_shapes=[pltpu.VMEM((8, _D), jnp.float32),
                        pltpu.VMEM((8, _D), jnp.float32)],
    )(xf, sc_part, ck, desc_emb, W_proj, gamma, beta, temp)
    return out


# speedup vs baseline: 1.4562x; 1.4562x over previous
import functools
import jax
import jax.numpy as jnp
from jax import lax
from jax.experimental import pallas as pl
from jax.experimental.pallas import tpu as pltpu
from jax.experimental.pallas import tpu_sc as plsc

_EPS = 1e-08
_B, _S, _D = 4, 2048, 768
_NUM_CLASSES_SEEN = 10
_TOP_K = 3
_N_DESC = 5

_TOTAL_ROWS = _B * _S
_TC_ROWS = 3 * _S
_SC_ROWS = _TOTAL_ROWS - _TC_ROWS
_NW = 16
_ROWS_PER_W = _SC_ROWS // _NW
_SC_CHUNK = 32
_SC_NCHUNK = _ROWS_PER_W // _SC_CHUNK
_VECS = _D // 16

_NSTEPS = 3
_ROWS_TC = _TC_ROWS // _NSTEPS


def _sc_body(x_hbm, out_hbm, buf0, buf1, acc, sem0, sem1):
    wid = lax.axis_index("s")
    base = _TC_ROWS + wid * _ROWS_PER_W

    bufs = (buf0, buf1)
    sems = (sem0, sem1)

    cps = {}
    for k in range(2):
        cps[k] = pltpu.async_copy(
            x_hbm.at[pl.ds(base + k * _SC_CHUNK, _SC_CHUNK)],
            bufs[k], sems[k])

    vals = tuple(jnp.zeros((16,), jnp.float32) for _ in range(_VECS))

    for k in range(_SC_NCHUNK):
        b = k % 2
        cps[k].wait()

        def _row(r, carry, buf=bufs[b]):
            return tuple(carry[j] + buf[r, pl.ds(16 * j, 16)]
                         for j in range(_VECS))

        vals = lax.fori_loop(0, _SC_CHUNK, _row, vals, unroll=2)
        if k + 2 < _SC_NCHUNK:
            cps[k + 2] = pltpu.async_copy(
                x_hbm.at[pl.ds(base + (k + 2) * _SC_CHUNK, _SC_CHUNK)],
                bufs[b], sems[b])

    for j in range(_VECS):
        acc[0, pl.ds(16 * j, 16)] = vals[j]

    pltpu.sync_copy(acc, out_hbm.at[pl.ds(wid, 1)])


@functools.partial(
    pl.kernel,
    out_type=jax.ShapeDtypeStruct((_NW, _D), jnp.float32),
    mesh=plsc.VectorSubcoreMesh(core_axis_name="c", subcore_axis_name="s",
                                num_cores=1),
    scratch_types=[
        pltpu.VMEM((_SC_CHUNK, _D), jnp.float32),
        pltpu.VMEM((_SC_CHUNK, _D), jnp.float32),
        pltpu.VMEM((1, _D), jnp.float32),
        pltpu.SemaphoreType.DMA,
        pltpu.SemaphoreType.DMA,
    ],
)
def _sc_partial_sums(x_hbm, out_hbm, buf0, buf1, acc, sem0, sem1):
    _sc_body(x_hbm, out_hbm, buf0, buf1, acc, sem0, sem1)


def _tc_body(x_ref, sc_ref, ck_ref, desc_ref, w_ref, g_ref, b_ref, t_ref,
             out_ref, acc_ref, dp_ref):
    i = pl.program_id(0)

    if _ROWS_TC <= _S:
        blks_per_batch = _S // _ROWS_TC
        partial = jnp.sum(x_ref[...], axis=0, keepdims=True)
        b = i // blks_per_batch

        @pl.when(i % blks_per_batch == 0)
        def _init():
            acc_ref[pl.ds(b, 1), :] = partial

        @pl.when(i % blks_per_batch != 0)
        def _accum():
            acc_ref[pl.ds(b, 1), :] = acc_ref[pl.ds(b, 1), :] + partial
    else:
        bpb = _ROWS_TC // _S
        partial = jnp.sum(x_ref[...].reshape(bpb, _S, _D), axis=1)
        for k in range(_NSTEPS):
            @pl.when(i == k)
            def _store(k=k):
                acc_ref[k * bpb:(k + 1) * bpb, :] = partial

    @pl.when(i == 0)
    def _dprime():
        dp_ref[0:_N_DESC, :] = jax.lax.dot_general(
            desc_ref[...], w_ref[...], (((1,), (1,)), ((), ())),
            preferred_element_type=jnp.float32)

    @pl.when(i == _NSTEPS - 1)
    def _epilogue():
        wpb = _NW // (_B - (_TC_ROWS // _S))
        sums = [acc_ref[0:(_TC_ROWS // _S), :]]
        for sb in range(_B - (_TC_ROWS // _S)):
            sums.append(jnp.sum(sc_ref[sb * wpb:(sb + 1) * wpb, :],
                                axis=0, keepdims=True))
        total = jnp.concatenate(sums, axis=0)

        mean = total * (1.0 / _S)
        xnorm = jnp.sqrt(jnp.sum(mean * mean, axis=1, keepdims=True))
        xn = mean / jnp.maximum(xnorm, _EPS)

        ck = ck_ref[...]
        cknorm = jnp.sqrt(jnp.sum(ck * ck, axis=1, keepdims=True))
        ckn = ck / jnp.maximum(cknorm, _EPS)

        sims = jax.lax.dot_general(
            xn, ckn, (((1,), (1,)), ((), ())),
            preferred_element_type=jnp.float32)

        t = t_ref[0, 0]

        col = jax.lax.broadcasted_iota(jnp.int32, (_B, _NUM_CLASSES_SEEN), 1)
        s = sims
        vals = []
        idxs = []
        for _ in range(_TOP_K):
            m = jnp.max(s, axis=1, keepdims=True)
            idx = jnp.min(jnp.where(s >= m, col, _NUM_CLASSES_SEEN + 1),
                          axis=1, keepdims=True)
            vals.append(m)
            idxs.append(idx)
            s = jnp.where(col == idx, -jnp.inf, s)

        exps = [jnp.exp((v - vals[0]) * t) for v in vals]
        denom = exps[0] + exps[1] + exps[2]
        ws = [e / denom for e in exps]

        dcol = jax.lax.broadcasted_iota(jnp.int32, (_B, _N_DESC), 1)
        dw = jnp.zeros((_B, _N_DESC), jnp.float32)
        for k in range(_TOP_K):
            didx = jax.lax.rem(idxs[k], _N_DESC)
            dw = dw + jnp.where(dcol == didx, ws[k], 0.0)

        proj = jax.lax.dot_general(
            dw, dp_ref[0:_N_DESC, :], (((1,), (0,)), ((), ())),
            preferred_element_type=jnp.float32)

        mu = jnp.mean(proj, axis=1, keepdims=True)
        ctr = proj - mu
        var = jnp.mean(ctr * ctr, axis=1, keepdims=True)
        ln = ctr * jax.lax.rsqrt(var + 1e-05) * g_ref[...] + b_ref[...]

        out_ref[:, 0, :] = ln


@jax.jit
def kernel(x_embed, prompt_key, task_key, desc_emb, W_proj, ln_gamma,
           ln_beta, temperature):
    del task_key
    xf = x_embed.reshape(_TOTAL_ROWS, _D)
    ck = prompt_key[:_NUM_CLASSES_SEEN]
    gamma = ln_gamma.reshape(1, _D)
    beta = ln_beta.reshape(1, _D)
    temp = temperature.reshape(1, 1)

    sc_part = _sc_partial_sums(xf)

    out = pl.pallas_call(
        _tc_body,
        grid=(_NSTEPS,),
        in_specs=[
            pl.BlockSpec((_ROWS_TC, _D), lambda i: (i, 0)),
            pl.BlockSpec((_NW, _D), lambda i: (0, 0)),
            pl.BlockSpec((_NUM_CLASSES_SEEN, _D), lambda i: (0, 0)),
            pl.BlockSpec((_N_DESC, _D), lambda i: (0, 0)),
            pl.BlockSpec((_D, _D), lambda i: (0, 0)),
            pl.BlockSpec((1, _D), lambda i: (0, 0)),
            pl.BlockSpec((1, _D), lambda i: (0, 0)),
            pl.BlockSpec((1, 1), lambda i: (0, 0)),
        ],
        out_specs=pl.BlockSpec((_B, 1, _D), lambda i: (0, 0, 0)),
        out_shape=jax.ShapeDtypeStruct((_B, 1, _D), jnp.float32),
        scratch_shapes=[pltpu.VMEM((8, _D), jnp.float32),
                        pltpu.VMEM((8, _D), jnp.float32)],
    )(xf, sc_part, ck, desc_emb, W_proj, gamma, beta, temp)
    return out


# speedup vs baseline: 1.5929x; 1.0939x over previous
import functools
import jax
import jax.numpy as jnp
from jax import lax
from jax.experimental import pallas as pl
from jax.experimental.pallas import tpu as pltpu
from jax.experimental.pallas import tpu_sc as plsc

_EPS = 1e-08
_B, _S, _D = 4, 2048, 768
_NUM_CLASSES_SEEN = 10
_TOP_K = 3
_N_DESC = 5

_SC_BATCH0 = 2
_N_SC_B = _B - _SC_BATCH0
_NW = 32
_WPB = _NW // _N_SC_B
_ROWS_PER_W = _S // _WPB
_SC_CHUNK = 32
_SC_NCHUNK = _ROWS_PER_W // _SC_CHUNK
_VECS = _D // 16

_NSTEPS = _SC_BATCH0


def _sc_body(x_hbm, out_hbm, buf0, buf1, acc, sem0, sem1):
    wid = lax.axis_index("s") * 2 + lax.axis_index("c")
    batch = _SC_BATCH0 + wid // _WPB
    row0 = (wid % _WPB) * _ROWS_PER_W

    bufs = (buf0, buf1)
    sems = (sem0, sem1)

    cps = {}
    for k in range(2):
        cps[k] = pltpu.async_copy(
            x_hbm.at[batch, pl.ds(row0 + k * _SC_CHUNK, _SC_CHUNK)],
            bufs[k], sems[k])

    vals = tuple(jnp.zeros((16,), jnp.float32) for _ in range(_VECS))

    for k in range(_SC_NCHUNK):
        b = k % 2
        cps[k].wait()

        def _row(r, carry, buf=bufs[b]):
            return tuple(carry[j] + buf[r, pl.ds(16 * j, 16)]
                         for j in range(_VECS))

        vals = lax.fori_loop(0, _SC_CHUNK, _row, vals, unroll=2)
        if k + 2 < _SC_NCHUNK:
            cps[k + 2] = pltpu.async_copy(
                x_hbm.at[batch, pl.ds(row0 + (k + 2) * _SC_CHUNK, _SC_CHUNK)],
                bufs[b], sems[b])

    for j in range(_VECS):
        acc[0, pl.ds(16 * j, 16)] = vals[j]

    pltpu.sync_copy(acc, out_hbm.at[pl.ds(wid, 1)])


@functools.partial(
    pl.kernel,
    out_type=jax.ShapeDtypeStruct((_NW, _D), jnp.float32),
    mesh=plsc.VectorSubcoreMesh(core_axis_name="c", subcore_axis_name="s"),
    scratch_types=[
        pltpu.VMEM((_SC_CHUNK, _D), jnp.float32),
        pltpu.VMEM((_SC_CHUNK, _D), jnp.float32),
        pltpu.VMEM((1, _D), jnp.float32),
        pltpu.SemaphoreType.DMA,
        pltpu.SemaphoreType.DMA,
    ],
)
def _sc_partial_sums(x_hbm, out_hbm, buf0, buf1, acc, sem0, sem1):
    _sc_body(x_hbm, out_hbm, buf0, buf1, acc, sem0, sem1)


def _tc_main_body(x_ref, desc_ref, w_ref, sums_ref, dp_ref):
    i = pl.program_id(0)
    partial = jnp.sum(x_ref[0], axis=0, keepdims=True)
    sums_ref[pl.ds(i, 1), :] = partial

    @pl.when(i == 0)
    def _dprime():
        dp_ref[0:_N_DESC, :] = jax.lax.dot_general(
            desc_ref[...], w_ref[...], (((1,), (1,)), ((), ())),
            preferred_element_type=jnp.float32)


def _tc_tail_body(sums_ref, sc_ref, dp_ref, ck_ref, g_ref, b_ref, t_ref,
                  out_ref):
    parts = [sums_ref[0:_SC_BATCH0, :]]
    for sb in range(_N_SC_B):
        parts.append(jnp.sum(sc_ref[sb * _WPB:(sb + 1) * _WPB, :],
                             axis=0, keepdims=True))
    total = jnp.concatenate(parts, axis=0)

    mean = total * (1.0 / _S)
    xnorm = jnp.sqrt(jnp.sum(mean * mean, axis=1, keepdims=True))
    xn = mean / jnp.maximum(xnorm, _EPS)

    ck = ck_ref[0:_NUM_CLASSES_SEEN, :]
    cknorm = jnp.sqrt(jnp.sum(ck * ck, axis=1, keepdims=True))
    ckn = ck / jnp.maximum(cknorm, _EPS)

    sims = jax.lax.dot_general(
        xn, ckn, (((1,), (1,)), ((), ())),
        preferred_element_type=jnp.float32)

    t = t_ref[0]

    col = jax.lax.broadcasted_iota(jnp.int32, (_B, _NUM_CLASSES_SEEN), 1)
    s = sims
    vals = []
    idxs = []
    for _ in range(_TOP_K):
        m = jnp.max(s, axis=1, keepdims=True)
        idx = jnp.min(jnp.where(s >= m, col, _NUM_CLASSES_SEEN + 1),
                      axis=1, keepdims=True)
        vals.append(m)
        idxs.append(idx)
        s = jnp.where(col == idx, -jnp.inf, s)

    exps = [jnp.exp((v - vals[0]) * t) for v in vals]
    denom = exps[0] + exps[1] + exps[2]
    ws = [e / denom for e in exps]

    dcol = jax.lax.broadcasted_iota(jnp.int32, (_B, _N_DESC), 1)
    dw = jnp.zeros((_B, _N_DESC), jnp.float32)
    for k in range(_TOP_K):
        didx = jax.lax.rem(idxs[k], _N_DESC)
        dw = dw + jnp.where(dcol == didx, ws[k], 0.0)

    proj = jax.lax.dot_general(
        dw, dp_ref[0:_N_DESC, :], (((1,), (0,)), ((), ())),
        preferred_element_type=jnp.float32)

    mu = jnp.mean(proj, axis=1, keepdims=True)
    ctr = proj - mu
    var = jnp.mean(ctr * ctr, axis=1, keepdims=True)
    ln = (ctr * jax.lax.rsqrt(var + 1e-05) * g_ref[...].reshape(1, _D)
          + b_ref[...].reshape(1, _D))

    out_ref[:, 0, :] = ln


@jax.jit
def kernel(x_embed, prompt_key, task_key, desc_emb, W_proj, ln_gamma,
           ln_beta, temperature):
    del task_key

    sc_part = _sc_partial_sums(x_embed)

    sums, dp = pl.pallas_call(
        _tc_main_body,
        grid=(_NSTEPS,),
        in_specs=[
            pl.BlockSpec((1, _S, _D), lambda i: (i, 0, 0)),
            pl.BlockSpec((_N_DESC, _D), lambda i: (0, 0)),
            pl.BlockSpec((_D, _D), lambda i: (0, 0)),
        ],
        out_specs=[pl.BlockSpec((8, _D), lambda i: (0, 0)),
                   pl.BlockSpec((8, _D), lambda i: (0, 0))],
        out_shape=[jax.ShapeDtypeStruct((8, _D), jnp.float32),
                   jax.ShapeDtypeStruct((8, _D), jnp.float32)],
    )(x_embed, desc_emb, W_proj)

    out = pl.pallas_call(
        _tc_tail_body,
        in_specs=[
            pl.BlockSpec((8, _D), lambda: (0, 0)),
            pl.BlockSpec((_NW, _D), lambda: (0, 0)),
            pl.BlockSpec((8, _D), lambda: (0, 0)),
            pl.BlockSpec((100, _D), lambda: (0, 0)),
            pl.BlockSpec((_D,), lambda: (0,)),
            pl.BlockSpec((_D,), lambda: (0,)),
            pl.BlockSpec(memory_space=pltpu.SMEM),
        ],
        out_specs=pl.BlockSpec((_B, 1, _D), lambda: (0, 0, 0)),
        out_shape=jax.ShapeDtypeStruct((_B, 1, _D), jnp.float32),
    )(sums, sc_part, dp, prompt_key, ln_gamma, ln_beta, temperature)
    return out


# speedup vs baseline: 4.2323x; 2.6571x over previous
import jax
import jax.numpy as jnp
from jax.experimental import pallas as pl
from jax.experimental.pallas import tpu as pltpu

_EPS = 1e-08
_B, _S, _D = 4, 2048, 768
_NUM_CLASSES_SEEN = 10
_TOP_K = 3
_N_DESC = 5
_S_CHUNK = 256
_NSTEPS = _S // _S_CHUNK


def _fused_body(x_ref, ck_ref, desc_ref, w_ref, g_ref, b_ref, t_ref,
                out_ref, acc_ref, dp_ref):
    i = pl.program_id(0)

    partial = jnp.sum(x_ref[...], axis=1)

    @pl.when(i == 0)
    def _init():
        acc_ref[0:_B, :] = partial
        dp_ref[0:_N_DESC, :] = jax.lax.dot_general(
            desc_ref[...], w_ref[...], (((1,), (1,)), ((), ())),
            preferred_element_type=jnp.float32)

    @pl.when(i > 0)
    def _accum():
        acc_ref[0:_B, :] = acc_ref[0:_B, :] + partial

    @pl.when(i == _NSTEPS - 1)
    def _epilogue():
        mean = acc_ref[0:_B, :] * (1.0 / _S)
        xnorm = jnp.sqrt(jnp.sum(mean * mean, axis=1, keepdims=True))
        xn = mean / jnp.maximum(xnorm, _EPS)

        ck = ck_ref[0:_NUM_CLASSES_SEEN, :]
        cknorm = jnp.sqrt(jnp.sum(ck * ck, axis=1, keepdims=True))
        ckn = ck / jnp.maximum(cknorm, _EPS)

        sims = jax.lax.dot_general(
            xn, ckn, (((1,), (1,)), ((), ())),
            preferred_element_type=jnp.float32)

        t = t_ref[0]

        col = jax.lax.broadcasted_iota(jnp.int32, (_B, _NUM_CLASSES_SEEN), 1)
        s = sims
        vals = []
        idxs = []
        for _ in range(_TOP_K):
            m = jnp.max(s, axis=1, keepdims=True)
            idx = jnp.min(jnp.where(s >= m, col, _NUM_CLASSES_SEEN + 1),
                          axis=1, keepdims=True)
            vals.append(m)
            idxs.append(idx)
            s = jnp.where(col == idx, -jnp.inf, s)

        exps = [jnp.exp((v - vals[0]) * t) for v in vals]
        denom = exps[0] + exps[1] + exps[2]
        ws = [e / denom for e in exps]

        dcol = jax.lax.broadcasted_iota(jnp.int32, (_B, _N_DESC), 1)
        dw = jnp.zeros((_B, _N_DESC), jnp.float32)
        for k in range(_TOP_K):
            didx = jax.lax.rem(idxs[k], _N_DESC)
            dw = dw + jnp.where(dcol == didx, ws[k], 0.0)

        proj = jax.lax.dot_general(
            dw, dp_ref[0:_N_DESC, :], (((1,), (0,)), ((), ())),
            preferred_element_type=jnp.float32)

        mu = jnp.mean(proj, axis=1, keepdims=True)
        ctr = proj - mu
        var = jnp.mean(ctr * ctr, axis=1, keepdims=True)
        ln = (ctr * jax.lax.rsqrt(var + 1e-05) * g_ref[...].reshape(1, _D)
              + b_ref[...].reshape(1, _D))

        out_ref[:, 0, :] = ln


@jax.jit
def kernel(x_embed, prompt_key, task_key, desc_emb, W_proj, ln_gamma,
           ln_beta, temperature):
    del task_key

    out = pl.pallas_call(
        _fused_body,
        grid=(_NSTEPS,),
        in_specs=[
            pl.BlockSpec((_B, _S_CHUNK, _D), lambda i: (0, i, 0)),
            pl.BlockSpec((16, _D), lambda i: (0, 0)),
            pl.BlockSpec((_N_DESC, _D), lambda i: (0, 0)),
            pl.BlockSpec((_D, _D), lambda i: (0, 0)),
            pl.BlockSpec((_D,), lambda i: (0,)),
            pl.BlockSpec((_D,), lambda i: (0,)),
            pl.BlockSpec(memory_space=pltpu.SMEM),
        ],
        out_specs=pl.BlockSpec((_B, 1, _D), lambda i: (0, 0, 0)),
        out_shape=jax.ShapeDtypeStruct((_B, 1, _D), jnp.float32),
        scratch_shapes=[pltpu.VMEM((8, _D), jnp.float32),
                        pltpu.VMEM((8, _D), jnp.float32)],
    )(x_embed, prompt_key, desc_emb, W_proj, ln_gamma, ln_beta, temperature)
    return out
